# baseline (speedup 1.0000x reference)
import functools

import jax
import jax.numpy as jnp
from jax import lax
from jax.experimental import pallas as pl
from jax.experimental.pallas import tpu as pltpu
from jax.experimental.pallas import tpu_sc as plsc

BATCH = 1024
LENGTH = 50
DIM = 32
BINS = DIM + DIM * DIM

NUM_CORES = 2
NUM_SUBCORES = 16
LANES = 16
STRIPES = 8
QUARTERS = 4
BINS_PER_W = BINS // QUARTERS
COLS_PER_W = BATCH // STRIPES
GROUPS = COLS_PER_W // LANES
PAIRS = LENGTH // 2


@functools.partial(
    pl.kernel,
    out_type=jax.ShapeDtypeStruct((BINS, BATCH), jnp.float32),
    mesh=plsc.VectorSubcoreMesh(core_axis_name="c", subcore_axis_name="s"),
    scratch_types=[
        pltpu.VMEM((LENGTH, COLS_PER_W), jnp.int32),
        pltpu.VMEM((BINS_PER_W, COLS_PER_W), jnp.float32),
    ],
    compiler_params=pltpu.CompilerParams(
        needs_layout_passes=False, disable_bounds_checks=True
    ),
)
def _ngram_counts_sc(in_hbm, out_hbm, tok_v, cnt_v):
    wid = lax.axis_index("s") * NUM_CORES + lax.axis_index("c")
    stripe = lax.rem(wid, STRIPES)
    quarter = lax.div(wid, STRIPES)
    col0 = stripe * COLS_PER_W
    lo = quarter * BINS_PER_W

    pltpu.sync_copy(in_hbm.at[:, pl.ds(col0, COLS_PER_W)], tok_v)

    zeros = jnp.zeros((LANES,), jnp.float32)

    @plsc.parallel_loop(0, BINS_PER_W // 2, unroll=4)
    def _zero_body(i):
        for j in range(2):
            for c in range(COLS_PER_W // LANES):
                cnt_v[i * 2 + j, pl.ds(c * LANES, LANES)] = zeros

    lane = lax.iota(jnp.int32, 16)
    ones = jnp.ones((LANES,), jnp.float32)
    lo_v = jnp.full((LANES,), 0, jnp.int32) + lo
    span = jnp.full((LANES,), BINS_PER_W, jnp.uint32)

    def _scatter(bin_v, col):
        local = bin_v - lo_v
        mask = plsc.bitcast(local, jnp.uint32) < span
        plsc.addupdate_scatter(cnt_v, [local, col], ones, mask=mask)

    @plsc.parallel_loop(0, GROUPS, unroll=2)
    def _group_body(g):
        coff = g * LANES
        col = coff + lane
        for p in range(PAIRS):
            a = tok_v[2 * p, pl.ds(coff, LANES)]
            b = tok_v[2 * p + 1, pl.ds(coff, LANES)]
            _scatter(a, col)
            _scatter(b, col)
            _scatter(DIM + a * DIM + b, col)

    pltpu.sync_copy(
        cnt_v, out_hbm.at[pl.ds(lo, BINS_PER_W), pl.ds(col0, COLS_PER_W)]
    )


def kernel(inputs):
    out_t = _ngram_counts_sc(inputs.T)
    return out_t.T


# speedup vs baseline: 1.0360x; 1.0360x over previous
import functools

import jax
import jax.numpy as jnp
from jax import lax
from jax.experimental import pallas as pl
from jax.experimental.pallas import tpu as pltpu
from jax.experimental.pallas import tpu_sc as plsc

BATCH = 1024
LENGTH = 50
DIM = 32
BINS = DIM + DIM * DIM

NUM_CORES = 2
NUM_SUBCORES = 16
LANES = 16
STRIPES = 8
QUARTERS = 4
BINS_PER_W = BINS // QUARTERS
COLS_PER_W = BATCH // STRIPES
GROUPS = COLS_PER_W // LANES
PAIRS = LENGTH // 2


@functools.partial(
    pl.kernel,
    out_type=jax.ShapeDtypeStruct((BINS, BATCH), jnp.float32),
    mesh=plsc.VectorSubcoreMesh(core_axis_name="c", subcore_axis_name="s"),
    scratch_types=[
        pltpu.VMEM((LENGTH, COLS_PER_W), jnp.int32),
        pltpu.VMEM((BINS_PER_W, COLS_PER_W), jnp.float32),
    ],
    compiler_params=pltpu.CompilerParams(
        needs_layout_passes=False, disable_bounds_checks=True
    ),
)
def _ngram_counts_sc(in_hbm, out_hbm, tok_v, cnt_v):
    wid = lax.axis_index("s") * NUM_CORES + lax.axis_index("c")
    stripe = lax.rem(wid, STRIPES)
    quarter = lax.div(wid, STRIPES)
    col0 = stripe * COLS_PER_W
    lo = quarter * BINS_PER_W

    pltpu.sync_copy(in_hbm.at[:, pl.ds(col0, COLS_PER_W)], tok_v)

    zeros = jnp.zeros((LANES,), jnp.float32)

    def _zero_body(i, carry):
        for j in range(8):
            for c in range(COLS_PER_W // LANES):
                cnt_v[i * 8 + j, pl.ds(c * LANES, LANES)] = zeros
        return carry

    lax.fori_loop(0, BINS_PER_W // 8, _zero_body, 0)

    lane = lax.iota(jnp.int32, 16)
    ones = jnp.ones((LANES,), jnp.float32)
    lo_v = jnp.full((LANES,), 0, jnp.int32) + lo
    span = jnp.full((LANES,), BINS_PER_W, jnp.uint32)

    def _scatter(bin_v, col):
        local = bin_v - lo_v
        mask = plsc.bitcast(local, jnp.uint32) < span
        plsc.addupdate_scatter(cnt_v, [local, col], ones, mask=mask)

    def _group_body(g, carry):
        coff = g * LANES
        col = coff + lane
        for p in range(PAIRS):
            a = tok_v[2 * p, pl.ds(coff, LANES)]
            b = tok_v[2 * p + 1, pl.ds(coff, LANES)]
            _scatter(a, col)
            _scatter(b, col)
            _scatter(DIM + a * DIM + b, col)
        return carry

    lax.fori_loop(0, GROUPS, _group_body, 0)

    pltpu.sync_copy(
        cnt_v, out_hbm.at[pl.ds(lo, BINS_PER_W), pl.ds(col0, COLS_PER_W)]
    )


def kernel(inputs):
    out_t = _ngram_counts_sc(inputs.T)
    return out_t.T


# speedup vs baseline: 1.0499x; 1.0134x over previous
import functools

import jax
import jax.numpy as jnp
from jax import lax
from jax.experimental import pallas as pl
from jax.experimental.pallas import tpu as pltpu
from jax.experimental.pallas import tpu_sc as plsc

BATCH = 1024
LENGTH = 50
DIM = 32
BINS = DIM + DIM * DIM

NUM_CORES = 2
NUM_SUBCORES = 16
LANES = 16
STRIPES = 8
QUARTERS = 4
BINS_PER_W = BINS // QUARTERS
COLS_PER_W = BATCH // STRIPES
GROUPS = COLS_PER_W // LANES
PAIRS = LENGTH // 2


@functools.partial(
    pl.kernel,
    out_type=jax.ShapeDtypeStruct((BINS, BATCH), jnp.float32),
    mesh=plsc.VectorSubcoreMesh(core_axis_name="c", subcore_axis_name="s"),
    scratch_types=[
        pltpu.VMEM((LENGTH, COLS_PER_W), jnp.int32),
        pltpu.VMEM((BINS_PER_W, COLS_PER_W), jnp.float32),
    ],
    compiler_params=pltpu.CompilerParams(
        needs_layout_passes=False, disable_bounds_checks=True
    ),
)
def _ngram_counts_sc(in_hbm, out_hbm, tok_v, cnt_v):
    wid = lax.axis_index("s") * NUM_CORES + lax.axis_index("c")
    stripe = lax.rem(wid, STRIPES)
    quarter = lax.div(wid, STRIPES)
    col0 = stripe * COLS_PER_W
    lo = quarter * BINS_PER_W

    pltpu.sync_copy(in_hbm.at[:, pl.ds(col0, COLS_PER_W)], tok_v)

    zeros = jnp.zeros((LANES,), jnp.float32)

    def _zero_body(i, carry):
        for j in range(2):
            for c in range(COLS_PER_W // LANES):
                cnt_v[i * 2 + j, pl.ds(c * LANES, LANES)] = zeros
        return carry

    lax.fori_loop(0, BINS_PER_W // 2, _zero_body, 0)

    lane = lax.iota(jnp.int32, 16)
    ones = jnp.ones((LANES,), jnp.float32)
    lo_v = jnp.full((LANES,), 0, jnp.int32) + lo
    span = jnp.full((LANES,), BINS_PER_W, jnp.uint32)
    uni_mask = lo_v < DIM

    def _scatter(bin_v, col):
        local = bin_v - lo_v
        mask = plsc.bitcast(local, jnp.uint32) < span
        plsc.addupdate_scatter(cnt_v, [local, col], ones, mask=mask)

    def _group_body(g, carry):
        coff = g * LANES
        col = coff + lane
        for p in range(PAIRS):
            a = tok_v[2 * p, pl.ds(coff, LANES)]
            b = tok_v[2 * p + 1, pl.ds(coff, LANES)]
            plsc.addupdate_scatter(cnt_v, [a, col], ones, mask=uni_mask)
            plsc.addupdate_scatter(cnt_v, [b, col], ones, mask=uni_mask)
            _scatter(DIM + a * DIM + b, col)
        return carry

    lax.fori_loop(0, GROUPS, _group_body, 0)

    pltpu.sync_copy(
        cnt_v, out_hbm.at[pl.ds(lo, BINS_PER_W), pl.ds(col0, COLS_PER_W)]
    )


def kernel(inputs):
    out_t = _ngram_counts_sc(inputs.T)
    return out_t.T


# speedup vs baseline: 1.0937x; 1.0417x over previous
import functools

import jax
import jax.numpy as jnp
from jax import lax
from jax.experimental import pallas as pl
from jax.experimental.pallas import tpu as pltpu
from jax.experimental.pallas import tpu_sc as plsc

BATCH = 1024
LENGTH = 50
DIM = 32
BINS = DIM + DIM * DIM

NUM_CORES = 2
NUM_SUBCORES = 16
LANES = 16
STRIPES = 8
QUARTERS = 4
BINS_PER_W = BINS // QUARTERS
COLS_PER_W = BATCH // STRIPES
GROUPS = COLS_PER_W // LANES
PAIRS = LENGTH // 2


@functools.partial(
    pl.kernel,
    out_type=jax.ShapeDtypeStruct((BINS, BATCH), jnp.float32),
    mesh=plsc.VectorSubcoreMesh(core_axis_name="c", subcore_axis_name="s"),
    scratch_types=[
        pltpu.VMEM((LENGTH, COLS_PER_W), jnp.int32),
        pltpu.VMEM((BINS_PER_W, COLS_PER_W), jnp.float32),
        pltpu.SemaphoreType.DMA,
    ],
    compiler_params=pltpu.CompilerParams(
        needs_layout_passes=False, disable_bounds_checks=True
    ),
)
def _ngram_counts_sc(in_hbm, out_hbm, tok_v, cnt_v, sem_in):
    wid = lax.axis_index("s") * NUM_CORES + lax.axis_index("c")
    stripe = lax.rem(wid, STRIPES)
    quarter = lax.div(wid, STRIPES)
    col0 = stripe * COLS_PER_W
    lo = quarter * BINS_PER_W

    stage = pltpu.async_copy(in_hbm.at[:, pl.ds(col0, COLS_PER_W)], tok_v, sem_in)

    zeros = jnp.zeros((LANES,), jnp.float32)

    def _zero_body(i, carry):
        for j in range(2):
            for c in range(COLS_PER_W // LANES):
                cnt_v[i * 2 + j, pl.ds(c * LANES, LANES)] = zeros
        return carry

    lax.fori_loop(0, BINS_PER_W // 2, _zero_body, 0)
    stage.wait()

    lane = lax.iota(jnp.int32, 16)
    ones = jnp.ones((LANES,), jnp.float32)
    lo_v = jnp.full((LANES,), 0, jnp.int32) + lo
    span = jnp.full((LANES,), BINS_PER_W, jnp.uint32)
    uni_mask = lo_v < DIM

    def _scatter(bin_v, col):
        local = bin_v - lo_v
        mask = plsc.bitcast(local, jnp.uint32) < span
        plsc.addupdate_scatter(cnt_v, [local, col], ones, mask=mask)

    def _group_body(g, carry):
        coff = g * LANES
        col = coff + lane
        for p in range(PAIRS):
            a = tok_v[2 * p, pl.ds(coff, LANES)]
            b = tok_v[2 * p + 1, pl.ds(coff, LANES)]
            plsc.addupdate_scatter(cnt_v, [a, col], ones, mask=uni_mask)
            plsc.addupdate_scatter(cnt_v, [b, col], ones, mask=uni_mask)
            _scatter(DIM + a * DIM + b, col)
        return carry

    lax.fori_loop(0, GROUPS, _group_body, 0)

    pltpu.sync_copy(
        cnt_v, out_hbm.at[pl.ds(lo, BINS_PER_W), pl.ds(col0, COLS_PER_W)]
    )


def kernel(inputs):
    out_t = _ngram_counts_sc(inputs.T)
    return out_t.T
